# Initial kernel scaffold; baseline (speedup 1.0000x reference)
import jax
import jax.numpy as jnp
from jax.experimental import pallas as pl


def kernel(context_words, emb_table, W1, b1, W2, b2):
    raise NotImplementedError("write your pallas kernel here")


# speedup vs baseline: 1.8567x; 1.8567x over previous
import functools

import jax
import jax.numpy as jnp
from jax import lax
from jax.experimental import pallas as pl
from jax.experimental.pallas import tpu as pltpu
from jax.experimental.pallas import tpu_sc as plsc

B, C, V, D, H, O = 1024, 50, 100000, 128, 4096, 4096
N_IDX = B * C

SC_CORES = 2
SC_SUBCORES = 16
NW = SC_CORES * SC_SUBCORES
B_PER_W = N_IDX // NW
GATHER_CHUNK = 400


def _sc_gather(emb_table, flat_idx):
    @functools.partial(
        pl.kernel,
        out_type=jax.ShapeDtypeStruct((N_IDX, D), jnp.float32),
        mesh=plsc.VectorSubcoreMesh(core_axis_name="c", subcore_axis_name="s"),
        scratch_types=[
            pltpu.VMEM((B_PER_W,), jnp.int32),
            pltpu.VMEM((GATHER_CHUNK, D), jnp.float32),
            pltpu.SemaphoreType.DMA,
        ],
    )
    def gather_kernel(table_hbm, idx_hbm, out_hbm, idx_v, rows_v, sem):
        wid = lax.axis_index("s") * SC_CORES + lax.axis_index("c")
        base = wid * B_PER_W
        pltpu.sync_copy(idx_hbm.at[pl.ds(base, B_PER_W)], idx_v)

        @pl.loop(0, B_PER_W, step=GATHER_CHUNK)
        def _(off):
            pltpu.async_copy(
                table_hbm.at[idx_v.at[pl.ds(off, GATHER_CHUNK)]], rows_v, sem
            ).wait()
            pltpu.sync_copy(rows_v, out_hbm.at[pl.ds(base + off, GATHER_CHUNK)])

    return gather_kernel(emb_table, flat_idx)


def _matmul_body(x_ref, w_ref, b_ref, o_ref, acc_ref, *, relu, out_dtype):
    k = pl.program_id(1)

    @pl.when(k == 0)
    def _():
        acc_ref[...] = jnp.zeros_like(acc_ref)

    acc_ref[...] += lax.dot_general(
        x_ref[...].astype(jnp.bfloat16),
        w_ref[...].astype(jnp.bfloat16),
        (((1,), (0,)), ((), ())),
        preferred_element_type=jnp.float32,
    )

    @pl.when(k == pl.num_programs(1) - 1)
    def _():
        r = acc_ref[...] + b_ref[...]
        if relu:
            r = jnp.maximum(r, 0.0)
        o_ref[...] = r.astype(out_dtype)


def _matmul(x, w, b, *, bn, bk, relu, out_dtype):
    m, kdim = x.shape
    n = w.shape[1]
    grid = (n // bn, kdim // bk)
    return pl.pallas_call(
        functools.partial(_matmul_body, relu=relu, out_dtype=out_dtype),
        grid=grid,
        in_specs=[
            pl.BlockSpec((m, bk), lambda i, j: (0, j)),
            pl.BlockSpec((bk, bn), lambda i, j: (j, i)),
            pl.BlockSpec((1, bn), lambda i, j: (0, i)),
        ],
        out_specs=pl.BlockSpec((m, bn), lambda i, j: (0, i)),
        out_shape=jax.ShapeDtypeStruct((m, n), out_dtype),
        scratch_shapes=[pltpu.VMEM((m, bn), jnp.float32)],
        compiler_params=pltpu.CompilerParams(
            dimension_semantics=("parallel", "arbitrary"),
        ),
    )(x, w, b.reshape(1, n))


def kernel(context_words, emb_table, W1, b1, W2, b2):
    flat_idx = context_words.reshape(-1).astype(jnp.int32)
    x = _sc_gather(emb_table, flat_idx).reshape(B, C * D)
    h = _matmul(x, W1, b1, bn=2048, bk=640, relu=True, out_dtype=jnp.bfloat16)
    out = _matmul(h, W2, b2, bn=2048, bk=1024, relu=False, out_dtype=jnp.float32)
    return out


# speedup vs baseline: 1.8570x; 1.0001x over previous
import functools

import jax
import jax.numpy as jnp
from jax import lax
from jax.experimental import pallas as pl
from jax.experimental.pallas import tpu as pltpu
from jax.experimental.pallas import tpu_sc as plsc

B, C, V, D, H, O = 1024, 50, 100000, 128, 4096, 4096
N_IDX = B * C

SC_CORES = 2
SC_SUBCORES = 16
NW = SC_CORES * SC_SUBCORES
B_PER_W = N_IDX // NW
GATHER_CHUNK = 400


def _sc_gather(emb_table, flat_idx):
    @functools.partial(
        pl.kernel,
        out_type=jax.ShapeDtypeStruct((N_IDX, D), jnp.float32),
        mesh=plsc.VectorSubcoreMesh(core_axis_name="c", subcore_axis_name="s"),
        scratch_types=[
            pltpu.VMEM((B_PER_W,), jnp.int32),
            pltpu.VMEM((GATHER_CHUNK, D), jnp.float32),
            pltpu.SemaphoreType.DMA,
        ],
    )
    def gather_kernel(table_hbm, idx_hbm, out_hbm, idx_v, rows_v, sem):
        wid = lax.axis_index("s") * SC_CORES + lax.axis_index("c")
        base = wid * B_PER_W
        pltpu.sync_copy(idx_hbm.at[pl.ds(base, B_PER_W)], idx_v)

        @pl.loop(0, B_PER_W, step=GATHER_CHUNK)
        def _(off):
            pltpu.async_copy(
                table_hbm.at[idx_v.at[pl.ds(off, GATHER_CHUNK)]], rows_v, sem
            ).wait()
            pltpu.sync_copy(rows_v, out_hbm.at[pl.ds(base + off, GATHER_CHUNK)])

    return gather_kernel(emb_table, flat_idx)


def _cast_body(x_ref, o_ref):
    o_ref[...] = x_ref[...].astype(jnp.bfloat16)


def _cast_bf16(x, *, bm=256):
    m, n = x.shape
    return pl.pallas_call(
        _cast_body,
        grid=(m // bm,),
        in_specs=[pl.BlockSpec((bm, n), lambda i: (i, 0))],
        out_specs=pl.BlockSpec((bm, n), lambda i: (i, 0)),
        out_shape=jax.ShapeDtypeStruct((m, n), jnp.bfloat16),
        compiler_params=pltpu.CompilerParams(
            dimension_semantics=("arbitrary",),
        ),
    )(x)


def _matmul_body(x_ref, w_ref, b_ref, o_ref, *, relu, out_dtype):
    r = lax.dot_general(
        x_ref[...],
        w_ref[...].astype(jnp.bfloat16),
        (((1,), (0,)), ((), ())),
        preferred_element_type=jnp.float32,
    )
    r = r + b_ref[...]
    if relu:
        r = jnp.maximum(r, 0.0)
    o_ref[...] = r.astype(out_dtype)


def _matmul(x, w, b, *, bn, relu, out_dtype):
    m, kdim = x.shape
    n = w.shape[1]
    return pl.pallas_call(
        functools.partial(_matmul_body, relu=relu, out_dtype=out_dtype),
        grid=(n // bn,),
        in_specs=[
            pl.BlockSpec((m, kdim), lambda i: (0, 0)),
            pl.BlockSpec((kdim, bn), lambda i: (0, i)),
            pl.BlockSpec((1, bn), lambda i: (0, i)),
        ],
        out_specs=pl.BlockSpec((m, bn), lambda i: (0, i)),
        out_shape=jax.ShapeDtypeStruct((m, n), out_dtype),
        compiler_params=pltpu.CompilerParams(
            dimension_semantics=("parallel",),
        ),
    )(x, w, b.reshape(1, n))


def kernel(context_words, emb_table, W1, b1, W2, b2):
    flat_idx = context_words.reshape(-1).astype(jnp.int32)
    x = _sc_gather(emb_table, flat_idx).reshape(B, C * D)
    xb = _cast_bf16(x)
    h = _matmul(xb, W1, b1, bn=512, relu=True, out_dtype=jnp.bfloat16)
    out = _matmul(h, W2, b2, bn=512, relu=False, out_dtype=jnp.float32)
    return out


# speedup vs baseline: 1.8964x; 1.0212x over previous
import functools

import jax
import jax.numpy as jnp
from jax import lax
from jax.experimental import pallas as pl
from jax.experimental.pallas import tpu as pltpu
from jax.experimental.pallas import tpu_sc as plsc

B, C, V, D, H, O = 1024, 50, 100000, 128, 4096, 4096
N_IDX = B * C

SC_CORES = 2
SC_SUBCORES = 16
NW = SC_CORES * SC_SUBCORES
B_PER_W = N_IDX // NW
GATHER_CHUNK = 400


def _sc_gather(emb_table, flat_idx):
    @functools.partial(
        pl.kernel,
        out_type=jax.ShapeDtypeStruct((N_IDX, D), jnp.float32),
        mesh=plsc.VectorSubcoreMesh(core_axis_name="c", subcore_axis_name="s"),
        scratch_types=[
            pltpu.VMEM((B_PER_W,), jnp.int32),
            pltpu.VMEM((GATHER_CHUNK, D), jnp.float32),
            pltpu.SemaphoreType.DMA,
        ],
    )
    def gather_kernel(table_hbm, idx_hbm, out_hbm, idx_v, rows_v, sem):
        wid = lax.axis_index("s") * SC_CORES + lax.axis_index("c")
        base = wid * B_PER_W
        pltpu.sync_copy(idx_hbm.at[pl.ds(base, B_PER_W)], idx_v)

        @pl.loop(0, B_PER_W, step=GATHER_CHUNK)
        def _(off):
            pltpu.async_copy(
                table_hbm.at[idx_v.at[pl.ds(off, GATHER_CHUNK)]], rows_v, sem
            ).wait()
            pltpu.sync_copy(rows_v, out_hbm.at[pl.ds(base + off, GATHER_CHUNK)])

    return gather_kernel(emb_table, flat_idx)


def _cast_body(x_ref, o_ref):
    o_ref[...] = x_ref[...].astype(jnp.bfloat16)


def _cast_assemble(x):
    return pl.pallas_call(
        _cast_body,
        grid=(C,),
        in_specs=[pl.BlockSpec((B, D), lambda c: (c, 0))],
        out_specs=pl.BlockSpec((B, D), lambda c: (0, c)),
        out_shape=jax.ShapeDtypeStruct((B, C * D), jnp.bfloat16),
        compiler_params=pltpu.CompilerParams(
            dimension_semantics=("arbitrary",),
        ),
    )(x)


def _matmul_body(x_ref, w_ref, b_ref, o_ref, *, relu, out_dtype):
    r = lax.dot_general(
        x_ref[...],
        w_ref[...].astype(jnp.bfloat16),
        (((1,), (0,)), ((), ())),
        preferred_element_type=jnp.float32,
    )
    r = r + b_ref[...]
    if relu:
        r = jnp.maximum(r, 0.0)
    o_ref[...] = r.astype(out_dtype)


def _matmul(x, w, b, *, bn, relu, out_dtype):
    m, kdim = x.shape
    n = w.shape[1]
    return pl.pallas_call(
        functools.partial(_matmul_body, relu=relu, out_dtype=out_dtype),
        grid=(n // bn,),
        in_specs=[
            pl.BlockSpec((m, kdim), lambda i: (0, 0)),
            pl.BlockSpec((kdim, bn), lambda i: (0, i)),
            pl.BlockSpec((1, bn), lambda i: (0, i)),
        ],
        out_specs=pl.BlockSpec((m, bn), lambda i: (0, i)),
        out_shape=jax.ShapeDtypeStruct((m, n), out_dtype),
        compiler_params=pltpu.CompilerParams(
            dimension_semantics=("parallel",),
        ),
    )(x, w, b.reshape(1, n))


def kernel(context_words, emb_table, W1, b1, W2, b2):
    flat_idx = context_words.T.reshape(-1).astype(jnp.int32)
    x = _sc_gather(emb_table, flat_idx)
    xb = _cast_assemble(x)
    h = _matmul(xb, W1, b1, bn=512, relu=True, out_dtype=jnp.bfloat16)
    out = _matmul(h, W2, b2, bn=512, relu=False, out_dtype=jnp.float32)
    return out


# speedup vs baseline: 1.8971x; 1.0004x over previous
import functools

import jax
import jax.numpy as jnp
from jax import lax
from jax.experimental import pallas as pl
from jax.experimental.pallas import tpu as pltpu
from jax.experimental.pallas import tpu_sc as plsc

B, C, V, D, H, O = 1024, 50, 100000, 128, 4096, 4096
N_IDX = B * C

SC_CORES = 2
SC_SUBCORES = 16
NW = SC_CORES * SC_SUBCORES
B_PER_W = N_IDX // NW
GATHER_CHUNK = 400


def _sc_gather(emb_table, flat_idx):
    @functools.partial(
        pl.kernel,
        out_type=jax.ShapeDtypeStruct((N_IDX, D), jnp.float32),
        mesh=plsc.VectorSubcoreMesh(core_axis_name="c", subcore_axis_name="s"),
        scratch_types=[
            pltpu.VMEM((B_PER_W,), jnp.int32),
            pltpu.VMEM((GATHER_CHUNK, D), jnp.float32),
            pltpu.SemaphoreType.DMA,
        ],
    )
    def gather_kernel(table_hbm, idx_hbm, out_hbm, idx_v, rows_v, sem):
        wid = lax.axis_index("s") * SC_CORES + lax.axis_index("c")
        base = wid * B_PER_W
        pltpu.sync_copy(idx_hbm.at[pl.ds(base, B_PER_W)], idx_v)

        @pl.loop(0, B_PER_W, step=GATHER_CHUNK)
        def _(off):
            pltpu.async_copy(
                table_hbm.at[idx_v.at[pl.ds(off, GATHER_CHUNK)]], rows_v, sem
            ).wait()
            pltpu.sync_copy(rows_v, out_hbm.at[pl.ds(base + off, GATHER_CHUNK)])

    return gather_kernel(emb_table, flat_idx)


def _cast_body(x_ref, o_ref):
    o_ref[...] = x_ref[...].astype(jnp.bfloat16)


def _cast_assemble(x):
    return pl.pallas_call(
        _cast_body,
        grid=(C,),
        in_specs=[pl.BlockSpec((B, D), lambda c: (c, 0))],
        out_specs=pl.BlockSpec((B, D), lambda c: (0, c)),
        out_shape=jax.ShapeDtypeStruct((B, C * D), jnp.bfloat16),
        compiler_params=pltpu.CompilerParams(
            dimension_semantics=("arbitrary",),
        ),
    )(x)


def _matmul_body(x_ref, w_ref, b_ref, o_ref, *, relu, out_dtype):
    r = lax.dot_general(
        x_ref[...],
        w_ref[...],
        (((1,), (0,)), ((), ())),
        preferred_element_type=jnp.float32,
    )
    r = r + b_ref[...]
    if relu:
        r = jnp.maximum(r, 0.0)
    o_ref[...] = r.astype(out_dtype)


def _matmul(x, w, b, *, bn, relu, out_dtype):
    m, kdim = x.shape
    n = w.shape[1]
    return pl.pallas_call(
        functools.partial(_matmul_body, relu=relu, out_dtype=out_dtype),
        grid=(n // bn,),
        in_specs=[
            pl.BlockSpec((m, kdim), lambda i: (0, 0)),
            pl.BlockSpec((kdim, bn), lambda i: (0, i)),
            pl.BlockSpec((1, bn), lambda i: (0, i)),
        ],
        out_specs=pl.BlockSpec((m, bn), lambda i: (0, i)),
        out_shape=jax.ShapeDtypeStruct((m, n), out_dtype),
        compiler_params=pltpu.CompilerParams(
            dimension_semantics=("parallel",),
        ),
    )(x, w, b.reshape(1, n))


def kernel(context_words, emb_table, W1, b1, W2, b2):
    flat_idx = context_words.T.reshape(-1).astype(jnp.int32)
    x = _sc_gather(emb_table, flat_idx)
    xb = _cast_assemble(x)
    h = _matmul(xb, W1, b1, bn=512, relu=True, out_dtype=jnp.bfloat16)
    out = _matmul(h, W2, b2, bn=512, relu=False, out_dtype=jnp.float32)
    return out
